# baseline (speedup 1.0000x reference)
import functools

import jax
import jax.numpy as jnp
import numpy as np
from jax.experimental import pallas as pl
from jax.experimental.pallas import tpu as pltpu
from jax.experimental.pallas import tpu_sc as plsc

D_MODEL = 1024
EXPERT_SIZE = 4096
N_EXP = 8
S = 2048
TOP_K = 2

T = 768
N_TILES = 13
TRASH = N_TILES
MAX_ROWS = N_TILES * T
F_BLK = 1024
N_FBLK = EXPERT_SIZE // F_BLK

CB = 128
N_CB = S // CB
_TRI = np.tril(np.ones((CB, CB), np.float32), -1)
_TRI8 = np.tril(np.ones((N_EXP, N_EXP), np.float32))
_TRI16 = np.tril(np.ones((16, 16), np.float32), -1)


def _gelu(h):
    return 0.5 * h * (1.0 + jax.lax.erf(h * 0.7071067811865476))


def _router_body(x_ref, wl_ref, bl_ref, wr_ref, br_ref, tri_ref, tri8_ref,
                 tri16_ref, rtok_ref, pn_ref, meta_ref):
    xx = x_ref[...]
    ll = jax.lax.dot_general(xx, wl_ref[...], (((1,), (1,)), ((), ())),
                             preferred_element_type=jnp.float32) + bl_ref[...]
    ll = ll - jnp.max(ll, axis=1, keepdims=True)
    el = jnp.exp(ll)
    lp = el / jnp.sum(el, axis=1, keepdims=True)
    shifted = jnp.concatenate([lp[:, 1:], jnp.zeros((S, 1), jnp.float32)],
                              axis=1)
    ew = lp + shifted
    rl = jax.lax.dot_general(xx, wr_ref[...], (((1,), (1,)), ((), ())),
                             preferred_element_type=jnp.float32)
    rl = rl + br_ref[...] + 0.1 * ew
    rl = rl - jnp.max(rl, axis=1, keepdims=True)
    er = jnp.exp(rl)
    rp = er / jnp.sum(er, axis=1, keepdims=True)

    lane = jax.lax.broadcasted_iota(jnp.int32, (S, N_EXP), 1)
    m1 = jnp.max(rp, axis=1, keepdims=True)
    i1 = jnp.argmax(rp, axis=1)[:, None].astype(jnp.int32)
    masked = jnp.where(lane == i1, -jnp.inf, rp)
    m2 = jnp.max(masked, axis=1, keepdims=True)
    i2 = jnp.argmax(masked, axis=1)[:, None].astype(jnp.int32)
    tot = m1 + m2
    p1 = m1 / tot
    p2 = m2 / tot
    oh1 = (lane == i1).astype(jnp.float32)
    oh2 = (lane == i2).astype(jnp.float32)
    ohb = oh1 + oh2

    tri = tri_ref[...]
    parts = []
    bsum = []
    for b in range(N_CB):
        blk = ohb[b * CB:(b + 1) * CB]
        parts.append(jax.lax.dot_general(
            tri, blk, (((1,), (0,)), ((), ())),
            preferred_element_type=jnp.float32))
        bsum.append(jnp.sum(blk, axis=0, keepdims=True))
    bsums = jnp.concatenate(bsum, axis=0)
    boff = jax.lax.dot_general(
        tri16_ref[...], bsums, (((1,), (0,)), ((), ())),
        preferred_element_type=jnp.float32)
    cum = jnp.concatenate(
        [parts[b] + boff[b][None, :] for b in range(N_CB)], axis=0)

    counts = jnp.sum(ohb, axis=0, keepdims=True)
    tiles_e = jnp.ceil(counts / T)
    tile_bound = jax.lax.dot_general(
        tiles_e, tri8_ref[...], (((1,), (1,)), ((), ())),
        preferred_element_type=jnp.float32)
    tile_off = tile_bound - tiles_e

    v = tile_off * jnp.float32(T) + cum
    dest1 = jnp.sum(oh1 * v, axis=1, keepdims=True)
    dest2 = jnp.sum(oh2 * v, axis=1, keepdims=True)
    rtok_ref[...] = jnp.concatenate([dest1, dest2], axis=1).astype(jnp.int32)
    pn_ref[...] = jnp.concatenate([p1, p2], axis=1)

    nt = jnp.sum(tiles_e)
    mlane = jax.lax.broadcasted_iota(jnp.int32, (8, 32), 1).astype(jnp.float32)
    mm = jnp.minimum(mlane, nt - 1.0)
    te = jnp.zeros((8, 32), jnp.float32)
    for e in range(N_EXP):
        te = te + (tile_bound[0, e] <= mm).astype(jnp.float32)
    tvalid = (mlane < nt).astype(jnp.float32)
    row = jax.lax.broadcasted_iota(jnp.int32, (8, 32), 0)
    meta = jnp.where(row == 0, te, jnp.where(row == 1, tvalid, 0.0))
    meta_ref[...] = meta.astype(jnp.int32)


def _router(xf, Wl, bl, Wr, br):
    return pl.pallas_call(
        _router_body,
        in_specs=[
            pl.BlockSpec((S, D_MODEL), lambda: (0, 0)),
            pl.BlockSpec((N_EXP, D_MODEL), lambda: (0, 0)),
            pl.BlockSpec((1, N_EXP), lambda: (0, 0)),
            pl.BlockSpec((N_EXP, D_MODEL), lambda: (0, 0)),
            pl.BlockSpec((1, N_EXP), lambda: (0, 0)),
            pl.BlockSpec((CB, CB), lambda: (0, 0)),
            pl.BlockSpec((N_EXP, N_EXP), lambda: (0, 0)),
            pl.BlockSpec((16, 16), lambda: (0, 0)),
        ],
        out_specs=[
            pl.BlockSpec((S, TOP_K), lambda: (0, 0)),
            pl.BlockSpec((S, TOP_K), lambda: (0, 0)),
            pl.BlockSpec((8, 32), lambda: (0, 0)),
        ],
        out_shape=[
            jax.ShapeDtypeStruct((S, TOP_K), jnp.int32),
            jax.ShapeDtypeStruct((S, TOP_K), jnp.float32),
            jax.ShapeDtypeStruct((8, 32), jnp.int32),
        ],
    )(xf, Wl, bl.reshape(1, N_EXP), Wr, br.reshape(1, N_EXP),
      jnp.asarray(_TRI), jnp.asarray(_TRI8), jnp.asarray(_TRI16))


NW = 32
TW = S // NW
CW = 32


def _dispatch_body(x_hbm, r0_hbm, r1_hbm, p0_hbm, p1_hbm, xg_hbm, rowp_hbm,
                   idx0_v, idx1_v, p0_v, p1_v, rows_v, sem0, sem1, sem2, sem3):
    wid = jax.lax.axis_index("s") * 2 + jax.lax.axis_index("c")
    base = wid * TW
    pltpu.sync_copy(r0_hbm.at[pl.ds(base, TW)], idx0_v)
    pltpu.sync_copy(r1_hbm.at[pl.ds(base, TW)], idx1_v)
    pltpu.sync_copy(p0_hbm.at[pl.ds(base, TW)], p0_v)
    pltpu.sync_copy(p1_hbm.at[pl.ds(base, TW)], p1_v)
    pltpu.sync_copy(x_hbm.at[pl.ds(base, TW)], rows_v)
    c0 = pltpu.async_copy(rows_v, xg_hbm.at[idx0_v], sem0)
    c1 = pltpu.async_copy(rows_v, xg_hbm.at[idx1_v], sem1)
    c2 = pltpu.async_copy(p0_v, rowp_hbm.at[idx0_v], sem2)
    c3 = pltpu.async_copy(p1_v, rowp_hbm.at[idx1_v], sem3)
    c0.wait()
    c1.wait()
    c2.wait()
    c3.wait()


def _dispatch_sc(xf, r0, r1, p0, p1):
    k = functools.partial(
        pl.kernel,
        mesh=plsc.VectorSubcoreMesh(core_axis_name="c", subcore_axis_name="s"),
        out_type=[
            jax.ShapeDtypeStruct((MAX_ROWS, D_MODEL), jnp.float32),
            jax.ShapeDtypeStruct((MAX_ROWS,), jnp.float32),
        ],
        scratch_types=[
            pltpu.VMEM((TW,), jnp.int32),
            pltpu.VMEM((TW,), jnp.int32),
            pltpu.VMEM((TW,), jnp.float32),
            pltpu.VMEM((TW,), jnp.float32),
            pltpu.VMEM((TW, D_MODEL), jnp.float32),
            pltpu.SemaphoreType.DMA,
            pltpu.SemaphoreType.DMA,
            pltpu.SemaphoreType.DMA,
            pltpu.SemaphoreType.DMA,
        ],
    )(_dispatch_body)
    return k(xf, r0, r1, p0, p1)


def _combine_body(y_hbm, r0_hbm, r1_hbm, a_hbm, b_hbm, idx0_v, idx1_v,
                  a_v, b_v, sem0, sem1):
    wid = jax.lax.axis_index("s") * 2 + jax.lax.axis_index("c")
    base = wid * TW
    for sub in range(TW // CW):
        b0 = base + sub * CW
        pltpu.sync_copy(r0_hbm.at[pl.ds(b0, CW)], idx0_v)
        pltpu.sync_copy(r1_hbm.at[pl.ds(b0, CW)], idx1_v)
        g0 = pltpu.async_copy(y_hbm.at[idx0_v], a_v, sem0)
        g1 = pltpu.async_copy(y_hbm.at[idx1_v], b_v, sem1)
        g0.wait()
        g1.wait()
        pltpu.sync_copy(a_v, a_hbm.at[pl.ds(b0, CW)])
        pltpu.sync_copy(b_v, b_hbm.at[pl.ds(b0, CW)])


def _combine_sc(y, r0, r1):
    k = functools.partial(
        pl.kernel,
        mesh=plsc.VectorSubcoreMesh(core_axis_name="c", subcore_axis_name="s"),
        out_type=[
            jax.ShapeDtypeStruct((S, D_MODEL), jnp.float32),
            jax.ShapeDtypeStruct((S, D_MODEL), jnp.float32),
        ],
        scratch_types=[
            pltpu.VMEM((CW,), jnp.int32),
            pltpu.VMEM((CW,), jnp.int32),
            pltpu.VMEM((CW, D_MODEL), jnp.float32),
            pltpu.VMEM((CW, D_MODEL), jnp.float32),
            pltpu.SemaphoreType.DMA,
            pltpu.SemaphoreType.DMA,
        ],
    )(_combine_body)
    return k(y, r0, r1)


def _grouped_body(tvalid_ref, te_ref, xg_ref, w1_ref, b1_ref, w2_ref,
                  b2_ref, rp_ref, out_ref):
    m = pl.program_id(0)
    f = pl.program_id(1)
    valid = tvalid_ref[m] == 1

    @pl.when(valid)
    def _compute():
        xx = xg_ref[...]
        h = jax.lax.dot_general(xx, w1_ref[0], (((1,), (1,)), ((), ())),
                                preferred_element_type=jnp.float32)
        g = _gelu(h + b1_ref[0, 0, 0][None, :])
        part = jax.lax.dot_general(g, w2_ref[0], (((1,), (1,)), ((), ())),
                                   preferred_element_type=jnp.float32)

        @pl.when(f == 0)
        def _():
            out_ref[...] = part + b2_ref[0, 0][None, :]

        @pl.when(f > 0)
        def _():
            out_ref[...] += part

        @pl.when(f == N_FBLK - 1)
        def _():
            out_ref[...] *= rp_ref[0, 0][:, None]


def _grouped_ffn(xg, W1, b1, W2, b2, rowp, tvalid, te_map):
    b1r = b1.reshape(N_EXP, N_FBLK, 1, F_BLK)
    b2r = b2.reshape(N_EXP, 1, D_MODEL)
    rpr = rowp.reshape(N_TILES, 1, T)

    def out_idx(m, f, tvalid_ref, te_ref):
        return (jnp.where(tvalid_ref[m] == 1, m, TRASH), 0)

    grid_spec = pltpu.PrefetchScalarGridSpec(
        num_scalar_prefetch=2,
        grid=(N_TILES, N_FBLK),
        in_specs=[
            pl.BlockSpec((T, D_MODEL), lambda m, f, tv, te: (m, 0)),
            pl.BlockSpec((1, F_BLK, D_MODEL), lambda m, f, tv, te: (te[m], f, 0)),
            pl.BlockSpec((1, 1, 1, F_BLK), lambda m, f, tv, te: (te[m], f, 0, 0)),
            pl.BlockSpec((1, D_MODEL, F_BLK), lambda m, f, tv, te: (te[m], 0, f)),
            pl.BlockSpec((1, 1, D_MODEL), lambda m, f, tv, te: (te[m], 0, 0)),
            pl.BlockSpec((1, 1, T), lambda m, f, tv, te: (m, 0, 0)),
        ],
        out_specs=pl.BlockSpec((T, D_MODEL), out_idx),
    )
    return pl.pallas_call(
        _grouped_body,
        grid_spec=grid_spec,
        out_shape=jax.ShapeDtypeStruct(((N_TILES + 1) * T, D_MODEL), jnp.float32),
    )(tvalid, te_map, xg, W1, b1r, W2, b2r, rpr)


@functools.partial(jax.jit, static_argnums=())
def kernel(x, Wl, bl, Wr, br, W1, b1, W2, b2):
    xf = x.reshape(S, D_MODEL)

    rtok, pn, meta = _router(xf, Wl, bl, Wr, br)
    te_map = meta[0, :N_TILES]
    tvalid = meta[1, :N_TILES]

    r0 = rtok[:, 0]
    r1 = rtok[:, 1]
    xg, rowp = _dispatch_sc(xf, r0, r1, pn[:, 0], pn[:, 1])
    y = _grouped_ffn(xg, W1, b1, W2, b2, rowp, tvalid, te_map)
    a, b = _combine_sc(y, r0, r1)
    return (a + b).reshape(x.shape)


# speedup vs baseline: 1.0528x; 1.0528x over previous
import functools

import jax
import jax.numpy as jnp
import numpy as np
from jax.experimental import pallas as pl
from jax.experimental.pallas import tpu as pltpu
from jax.experimental.pallas import tpu_sc as plsc

D_MODEL = 1024
EXPERT_SIZE = 4096
N_EXP = 8
S = 2048
TOP_K = 2

T = 768
N_TILES = 13
TRASH = N_TILES
MAX_ROWS = N_TILES * T
F_BLK = 2048
N_FBLK = EXPERT_SIZE // F_BLK

CB = 128
N_CB = S // CB
_TRI = np.tril(np.ones((CB, CB), np.float32), -1)
_TRI8 = np.tril(np.ones((N_EXP, N_EXP), np.float32))
_TRI16 = np.tril(np.ones((16, 16), np.float32), -1)


def _gelu(h):
    return 0.5 * h * (1.0 + jax.lax.erf(h * 0.7071067811865476))


def _router_body(x_ref, wl_ref, bl_ref, wr_ref, br_ref, tri_ref, tri8_ref,
                 tri16_ref, rtok_ref, pn_ref, meta_ref):
    xx = x_ref[...]
    ll = jax.lax.dot_general(xx, wl_ref[...], (((1,), (1,)), ((), ())),
                             preferred_element_type=jnp.float32) + bl_ref[...]
    ll = ll - jnp.max(ll, axis=1, keepdims=True)
    el = jnp.exp(ll)
    lp = el / jnp.sum(el, axis=1, keepdims=True)
    shifted = jnp.concatenate([lp[:, 1:], jnp.zeros((S, 1), jnp.float32)],
                              axis=1)
    ew = lp + shifted
    rl = jax.lax.dot_general(xx, wr_ref[...], (((1,), (1,)), ((), ())),
                             preferred_element_type=jnp.float32)
    rl = rl + br_ref[...] + 0.1 * ew
    rl = rl - jnp.max(rl, axis=1, keepdims=True)
    er = jnp.exp(rl)
    rp = er / jnp.sum(er, axis=1, keepdims=True)

    lane = jax.lax.broadcasted_iota(jnp.int32, (S, N_EXP), 1)
    m1 = jnp.max(rp, axis=1, keepdims=True)
    i1 = jnp.argmax(rp, axis=1)[:, None].astype(jnp.int32)
    masked = jnp.where(lane == i1, -jnp.inf, rp)
    m2 = jnp.max(masked, axis=1, keepdims=True)
    i2 = jnp.argmax(masked, axis=1)[:, None].astype(jnp.int32)
    tot = m1 + m2
    p1 = m1 / tot
    p2 = m2 / tot
    oh1 = (lane == i1).astype(jnp.float32)
    oh2 = (lane == i2).astype(jnp.float32)
    ohb = oh1 + oh2

    tri = tri_ref[...]
    parts = []
    bsum = []
    for b in range(N_CB):
        blk = ohb[b * CB:(b + 1) * CB]
        parts.append(jax.lax.dot_general(
            tri, blk, (((1,), (0,)), ((), ())),
            preferred_element_type=jnp.float32))
        bsum.append(jnp.sum(blk, axis=0, keepdims=True))
    bsums = jnp.concatenate(bsum, axis=0)
    boff = jax.lax.dot_general(
        tri16_ref[...], bsums, (((1,), (0,)), ((), ())),
        preferred_element_type=jnp.float32)
    cum = jnp.concatenate(
        [parts[b] + boff[b][None, :] for b in range(N_CB)], axis=0)

    counts = jnp.sum(ohb, axis=0, keepdims=True)
    tiles_e = jnp.ceil(counts / T)
    tile_bound = jax.lax.dot_general(
        tiles_e, tri8_ref[...], (((1,), (1,)), ((), ())),
        preferred_element_type=jnp.float32)
    tile_off = tile_bound - tiles_e

    v = tile_off * jnp.float32(T) + cum
    dest1 = jnp.sum(oh1 * v, axis=1, keepdims=True)
    dest2 = jnp.sum(oh2 * v, axis=1, keepdims=True)
    rtok_ref[...] = jnp.concatenate([dest1, dest2], axis=1).astype(jnp.int32)
    pn_ref[...] = jnp.concatenate([p1, p2], axis=1)

    nt = jnp.sum(tiles_e)
    mlane = jax.lax.broadcasted_iota(jnp.int32, (8, 32), 1).astype(jnp.float32)
    mm = jnp.minimum(mlane, nt - 1.0)
    te = jnp.zeros((8, 32), jnp.float32)
    for e in range(N_EXP):
        te = te + (tile_bound[0, e] <= mm).astype(jnp.float32)
    tvalid = (mlane < nt).astype(jnp.float32)
    row = jax.lax.broadcasted_iota(jnp.int32, (8, 32), 0)
    meta = jnp.where(row == 0, te, jnp.where(row == 1, tvalid, 0.0))
    meta_ref[...] = meta.astype(jnp.int32)


def _router(xf, Wl, bl, Wr, br):
    return pl.pallas_call(
        _router_body,
        in_specs=[
            pl.BlockSpec((S, D_MODEL), lambda: (0, 0)),
            pl.BlockSpec((N_EXP, D_MODEL), lambda: (0, 0)),
            pl.BlockSpec((1, N_EXP), lambda: (0, 0)),
            pl.BlockSpec((N_EXP, D_MODEL), lambda: (0, 0)),
            pl.BlockSpec((1, N_EXP), lambda: (0, 0)),
            pl.BlockSpec((CB, CB), lambda: (0, 0)),
            pl.BlockSpec((N_EXP, N_EXP), lambda: (0, 0)),
            pl.BlockSpec((16, 16), lambda: (0, 0)),
        ],
        out_specs=[
            pl.BlockSpec((S, TOP_K), lambda: (0, 0)),
            pl.BlockSpec((S, TOP_K), lambda: (0, 0)),
            pl.BlockSpec((8, 32), lambda: (0, 0)),
        ],
        out_shape=[
            jax.ShapeDtypeStruct((S, TOP_K), jnp.int32),
            jax.ShapeDtypeStruct((S, TOP_K), jnp.float32),
            jax.ShapeDtypeStruct((8, 32), jnp.int32),
        ],
    )(xf, Wl, bl.reshape(1, N_EXP), Wr, br.reshape(1, N_EXP),
      jnp.asarray(_TRI), jnp.asarray(_TRI8), jnp.asarray(_TRI16))


NW = 32
TW = S // NW
CW = 32


def _dispatch_body(x_hbm, r0_hbm, r1_hbm, p0_hbm, p1_hbm, xg_hbm, rowp_hbm,
                   idx0_v, idx1_v, p0_v, p1_v, rows_v, sem0, sem1, sem2, sem3):
    wid = jax.lax.axis_index("s") * 2 + jax.lax.axis_index("c")
    base = wid * TW
    pltpu.sync_copy(r0_hbm.at[pl.ds(base, TW)], idx0_v)
    pltpu.sync_copy(r1_hbm.at[pl.ds(base, TW)], idx1_v)
    pltpu.sync_copy(p0_hbm.at[pl.ds(base, TW)], p0_v)
    pltpu.sync_copy(p1_hbm.at[pl.ds(base, TW)], p1_v)
    pltpu.sync_copy(x_hbm.at[pl.ds(base, TW)], rows_v)
    c0 = pltpu.async_copy(rows_v, xg_hbm.at[idx0_v], sem0)
    c1 = pltpu.async_copy(rows_v, xg_hbm.at[idx1_v], sem1)
    c2 = pltpu.async_copy(p0_v, rowp_hbm.at[idx0_v], sem2)
    c3 = pltpu.async_copy(p1_v, rowp_hbm.at[idx1_v], sem3)
    c0.wait()
    c1.wait()
    c2.wait()
    c3.wait()


def _dispatch_sc(xf, r0, r1, p0, p1):
    k = functools.partial(
        pl.kernel,
        mesh=plsc.VectorSubcoreMesh(core_axis_name="c", subcore_axis_name="s"),
        out_type=[
            jax.ShapeDtypeStruct((MAX_ROWS, D_MODEL), jnp.float32),
            jax.ShapeDtypeStruct((MAX_ROWS,), jnp.float32),
        ],
        scratch_types=[
            pltpu.VMEM((TW,), jnp.int32),
            pltpu.VMEM((TW,), jnp.int32),
            pltpu.VMEM((TW,), jnp.float32),
            pltpu.VMEM((TW,), jnp.float32),
            pltpu.VMEM((TW, D_MODEL), jnp.float32),
            pltpu.SemaphoreType.DMA,
            pltpu.SemaphoreType.DMA,
            pltpu.SemaphoreType.DMA,
            pltpu.SemaphoreType.DMA,
        ],
    )(_dispatch_body)
    return k(xf, r0, r1, p0, p1)


def _combine_body(y_hbm, r0_hbm, r1_hbm, a_hbm, b_hbm, idx0_v, idx1_v,
                  a_v, b_v, sem0, sem1):
    wid = jax.lax.axis_index("s") * 2 + jax.lax.axis_index("c")
    base = wid * TW
    for sub in range(TW // CW):
        b0 = base + sub * CW
        pltpu.sync_copy(r0_hbm.at[pl.ds(b0, CW)], idx0_v)
        pltpu.sync_copy(r1_hbm.at[pl.ds(b0, CW)], idx1_v)
        g0 = pltpu.async_copy(y_hbm.at[idx0_v], a_v, sem0)
        g1 = pltpu.async_copy(y_hbm.at[idx1_v], b_v, sem1)
        g0.wait()
        g1.wait()
        pltpu.sync_copy(a_v, a_hbm.at[pl.ds(b0, CW)])
        pltpu.sync_copy(b_v, b_hbm.at[pl.ds(b0, CW)])


def _combine_sc(y, r0, r1):
    k = functools.partial(
        pl.kernel,
        mesh=plsc.VectorSubcoreMesh(core_axis_name="c", subcore_axis_name="s"),
        out_type=[
            jax.ShapeDtypeStruct((S, D_MODEL), jnp.float32),
            jax.ShapeDtypeStruct((S, D_MODEL), jnp.float32),
        ],
        scratch_types=[
            pltpu.VMEM((CW,), jnp.int32),
            pltpu.VMEM((CW,), jnp.int32),
            pltpu.VMEM((CW, D_MODEL), jnp.float32),
            pltpu.VMEM((CW, D_MODEL), jnp.float32),
            pltpu.SemaphoreType.DMA,
            pltpu.SemaphoreType.DMA,
        ],
    )(_combine_body)
    return k(y, r0, r1)


def _grouped_body(tvalid_ref, te_ref, xg_ref, w1_ref, b1_ref, w2_ref,
                  b2_ref, rp_ref, out_ref):
    m = pl.program_id(0)
    f = pl.program_id(1)
    valid = tvalid_ref[m] == 1

    @pl.when(valid)
    def _compute():
        xx = xg_ref[...]
        h = jax.lax.dot_general(xx, w1_ref[0], (((1,), (1,)), ((), ())),
                                preferred_element_type=jnp.float32)
        g = _gelu(h + b1_ref[0, 0, 0][None, :])
        part = jax.lax.dot_general(g, w2_ref[0], (((1,), (1,)), ((), ())),
                                   preferred_element_type=jnp.float32)

        @pl.when(f == 0)
        def _():
            out_ref[...] = part + b2_ref[0, 0][None, :]

        @pl.when(f > 0)
        def _():
            out_ref[...] += part

        @pl.when(f == N_FBLK - 1)
        def _():
            out_ref[...] *= rp_ref[0, 0][:, None]


def _grouped_ffn(xg, W1, b1, W2, b2, rowp, tvalid, te_map):
    b1r = b1.reshape(N_EXP, N_FBLK, 1, F_BLK)
    b2r = b2.reshape(N_EXP, 1, D_MODEL)
    rpr = rowp.reshape(N_TILES, 1, T)

    def out_idx(m, f, tvalid_ref, te_ref):
        return (jnp.where(tvalid_ref[m] == 1, m, TRASH), 0)

    grid_spec = pltpu.PrefetchScalarGridSpec(
        num_scalar_prefetch=2,
        grid=(N_TILES, N_FBLK),
        in_specs=[
            pl.BlockSpec((T, D_MODEL), lambda m, f, tv, te: (m, 0)),
            pl.BlockSpec((1, F_BLK, D_MODEL), lambda m, f, tv, te: (te[m], f, 0)),
            pl.BlockSpec((1, 1, 1, F_BLK), lambda m, f, tv, te: (te[m], f, 0, 0)),
            pl.BlockSpec((1, D_MODEL, F_BLK), lambda m, f, tv, te: (te[m], 0, f)),
            pl.BlockSpec((1, 1, D_MODEL), lambda m, f, tv, te: (te[m], 0, 0)),
            pl.BlockSpec((1, 1, T), lambda m, f, tv, te: (m, 0, 0)),
        ],
        out_specs=pl.BlockSpec((T, D_MODEL), out_idx),
    )
    return pl.pallas_call(
        _grouped_body,
        grid_spec=grid_spec,
        out_shape=jax.ShapeDtypeStruct(((N_TILES + 1) * T, D_MODEL), jnp.float32),
    )(tvalid, te_map, xg, W1, b1r, W2, b2r, rpr)


@functools.partial(jax.jit, static_argnums=())
def kernel(x, Wl, bl, Wr, br, W1, b1, W2, b2):
    xf = x.reshape(S, D_MODEL)

    rtok, pn, meta = _router(xf, Wl, bl, Wr, br)
    te_map = meta[0, :N_TILES]
    tvalid = meta[1, :N_TILES]

    r0 = rtok[:, 0]
    r1 = rtok[:, 1]
    xg, rowp = _dispatch_sc(xf, r0, r1, pn[:, 0], pn[:, 1])
    y = _grouped_ffn(xg, W1, b1, W2, b2, rowp, tvalid, te_map)
    a, b = _combine_sc(y, r0, r1)
    return (a + b).reshape(x.shape)
